# baseline (speedup 1.0000x reference)
import functools

import jax
import jax.numpy as jnp
from jax import lax
from jax.experimental import pallas as pl
from jax.experimental.pallas import tpu as pltpu
from jax.experimental.pallas import tpu_sc as plsc

B = 16
S = 2048
HALF = S
LANES = 16
START_VALUE = 101
END_VALUE = 102
PAD_VALUE = 0
BUDGET = S - 3


def _body(seq_a, seq_b, len_a, len_b, tok_out, seg_out,
          src_v, lena_v, lenb_v, tok_v, seg_v, sem_seq, sem_len, sem_out):
    row = lax.axis_index("s")
    half = lax.axis_index("c")

    cpy_a = pltpu.async_copy(seq_a.at[row], src_v.at[pl.ds(0, S)], sem_seq)
    cpy_b = pltpu.async_copy(seq_b.at[row], src_v.at[pl.ds(S, S)], sem_seq)
    cpy_la = pltpu.async_copy(len_a, lena_v, sem_len)
    cpy_lb = pltpu.async_copy(len_b, lenb_v, sem_len)
    cpy_la.wait()
    cpy_lb.wait()

    rowv = jnp.full((LANES,), row, jnp.int32)
    l1 = jnp.minimum(plsc.load_gather(lena_v, [rowv]), S)
    l2 = jnp.minimum(plsc.load_gather(lenb_v, [rowv]), S)
    a1 = jnp.minimum(l1, jnp.maximum((BUDGET + 1) // 2, BUDGET - l2))
    a2 = jnp.minimum(l2, jnp.maximum(BUDGET // 2, BUDGET - l1))
    e1 = 1 + a1
    s2 = 2 + a1
    e2 = s2 + a2
    s2m = s2 - S

    base = half * HALF
    iota = lax.iota(jnp.int32, 16)
    zeros = jnp.zeros((LANES,), jnp.int32)
    ones = jnp.full((LANES,), 1, jnp.int32)

    cpy_a.wait()
    cpy_b.wait()

    def _loop(i, _):
        pos = iota + (base + i * LANES)
        m2 = pos >= s2
        idx = jnp.where(m2, pos - s2m, pos - 1)
        g = plsc.load_gather(src_v, [jnp.maximum(idx, zeros)])
        gt2 = pos > e2
        tok = jnp.where(gt2, zeros, g)
        seg = jnp.where(m2 & jnp.logical_not(gt2), ones, zeros)
        tok_v[pl.ds(i * LANES, LANES)] = tok
        seg_v[pl.ds(i * LANES, LANES)] = seg
        return _

    lax.fori_loop(0, HALF // LANES, _loop, 0)

    cand = jnp.where(iota == 0, zeros, jnp.where(iota == 1, e1, e2))
    val = jnp.where(iota == 0, jnp.full((LANES,), START_VALUE, jnp.int32),
                    jnp.full((LANES,), END_VALUE, jnp.int32))
    rel = cand - base
    mask = (iota < 3) & (rel >= 0) & (rel < HALF)
    plsc.store_scatter(tok_v, [jnp.clip(rel, 0, HALF - 1)], val, mask=mask)

    out_a = pltpu.async_copy(tok_v, tok_out.at[row, pl.ds(base, HALF)], sem_out)
    out_b = pltpu.async_copy(seg_v, seg_out.at[row, pl.ds(base, HALF)], sem_out)
    out_a.wait()
    out_b.wait()


@jax.jit
def kernel(seq_a, seq_b, len_a, len_b):
    mesh = plsc.VectorSubcoreMesh(core_axis_name="c", subcore_axis_name="s",
                                  num_cores=1, num_subcores=16)
    out_type = (jax.ShapeDtypeStruct((B, S), jnp.int32),
                jax.ShapeDtypeStruct((B, S), jnp.int32))
    scratch = [
        pltpu.VMEM((2 * S,), jnp.int32),
        pltpu.VMEM((B,), jnp.int32),
        pltpu.VMEM((B,), jnp.int32),
        pltpu.VMEM((HALF,), jnp.int32),
        pltpu.VMEM((HALF,), jnp.int32),
        pltpu.SemaphoreType.DMA,
        pltpu.SemaphoreType.DMA,
        pltpu.SemaphoreType.DMA,
    ]
    f = pl.kernel(_body, out_type=out_type, mesh=mesh, scratch_types=scratch,
                  compiler_params=pltpu.CompilerParams(needs_layout_passes=False))
    tok, seg = f(seq_a.astype(jnp.int32), seq_b.astype(jnp.int32),
                 len_a.astype(jnp.int32), len_b.astype(jnp.int32))
    return tok, seg


# speedup vs baseline: 1.0092x; 1.0092x over previous
import jax
import jax.numpy as jnp
from jax import lax
from jax.experimental import pallas as pl
from jax.experimental.pallas import tpu as pltpu
from jax.experimental.pallas import tpu_sc as plsc

B = 16
S = 2048
HALFW = S // 2
LANES = 16
START_VALUE = 101
END_VALUE = 102
PAD_VALUE = 0
BUDGET = S - 3


def _body(seq_a, seq_b, len_a, len_b, tok_out, seg_out,
          src_v, lena_v, lenb_v, tok_v, seg_v, sem_lo, sem_hi, sem_len, sem_out):
    row = lax.axis_index("s")

    cp_alo = pltpu.async_copy(seq_a.at[row, pl.ds(0, HALFW)],
                              src_v.at[pl.ds(0, HALFW)], sem_lo)
    cp_blo = pltpu.async_copy(seq_b.at[row, pl.ds(0, HALFW)],
                              src_v.at[pl.ds(S, HALFW)], sem_lo)
    cp_ahi = pltpu.async_copy(seq_a.at[row, pl.ds(HALFW, HALFW)],
                              src_v.at[pl.ds(HALFW, HALFW)], sem_hi)
    cp_bhi = pltpu.async_copy(seq_b.at[row, pl.ds(HALFW, HALFW)],
                              src_v.at[pl.ds(S + HALFW, HALFW)], sem_hi)
    cp_la = pltpu.async_copy(len_a, lena_v, sem_len)
    cp_lb = pltpu.async_copy(len_b, lenb_v, sem_len)
    cp_la.wait()
    cp_lb.wait()

    rowv = jnp.full((LANES,), row, jnp.int32)
    l1 = jnp.minimum(plsc.load_gather(lena_v, [rowv]), S)
    l2 = jnp.minimum(plsc.load_gather(lenb_v, [rowv]), S)
    a1 = jnp.minimum(l1, jnp.maximum((BUDGET + 1) // 2, BUDGET - l2))
    a2 = jnp.minimum(l2, jnp.maximum(BUDGET // 2, BUDGET - l1))
    e1 = 1 + a1
    s2 = 2 + a1
    e2 = s2 + a2
    s2m = s2 - S

    iota = lax.iota(jnp.int32, 16)
    zeros = jnp.zeros((LANES,), jnp.int32)
    ones = jnp.full((LANES,), 1, jnp.int32)

    cand = jnp.where(iota == 0, zeros, jnp.where(iota == 1, e1, e2))
    val = jnp.where(iota == 0, jnp.full((LANES,), START_VALUE, jnp.int32),
                    jnp.full((LANES,), END_VALUE, jnp.int32))

    out_copies = []
    for chunk, in_copies in ((0, (cp_alo, cp_blo)), (1, (cp_ahi, cp_bhi))):
        for c in in_copies:
            c.wait()
        lo = chunk * (HALFW // LANES)

        @plsc.parallel_loop(lo, lo + HALFW // LANES, unroll=4)
        def _loop(i):
            pos = iota + i * LANES
            m2 = pos >= s2
            idx = jnp.where(m2, pos - s2m, pos - 1)
            g = plsc.load_gather(src_v, [jnp.maximum(idx, zeros)])
            gt2 = pos > e2
            tok = jnp.where(gt2, zeros, g)
            seg = jnp.where(m2 & jnp.logical_not(gt2), ones, zeros)
            tok_v[pl.ds(i * LANES, LANES)] = tok
            seg_v[pl.ds(i * LANES, LANES)] = seg

        off = chunk * HALFW
        rel = cand - off
        mask = (iota < 3) & (rel >= 0) & (rel < HALFW)
        plsc.store_scatter(tok_v.at[pl.ds(off, HALFW)],
                           [jnp.clip(rel, 0, HALFW - 1)], val, mask=mask)
        out_copies.append(pltpu.async_copy(
            tok_v.at[pl.ds(off, HALFW)],
            tok_out.at[row, pl.ds(off, HALFW)], sem_out))
        out_copies.append(pltpu.async_copy(
            seg_v.at[pl.ds(off, HALFW)],
            seg_out.at[row, pl.ds(off, HALFW)], sem_out))

    for c in out_copies:
        c.wait()


@jax.jit
def kernel(seq_a, seq_b, len_a, len_b):
    mesh = plsc.VectorSubcoreMesh(core_axis_name="c", subcore_axis_name="s",
                                  num_cores=1, num_subcores=16)
    out_type = (jax.ShapeDtypeStruct((B, S), jnp.int32),
                jax.ShapeDtypeStruct((B, S), jnp.int32))
    scratch = [
        pltpu.VMEM((2 * S,), jnp.int32),
        pltpu.VMEM((B,), jnp.int32),
        pltpu.VMEM((B,), jnp.int32),
        pltpu.VMEM((S,), jnp.int32),
        pltpu.VMEM((S,), jnp.int32),
        pltpu.SemaphoreType.DMA,
        pltpu.SemaphoreType.DMA,
        pltpu.SemaphoreType.DMA,
        pltpu.SemaphoreType.DMA,
    ]
    f = pl.kernel(_body, out_type=out_type, mesh=mesh, scratch_types=scratch,
                  compiler_params=pltpu.CompilerParams(needs_layout_passes=False))
    tok, seg = f(seq_a.astype(jnp.int32), seq_b.astype(jnp.int32),
                 len_a.astype(jnp.int32), len_b.astype(jnp.int32))
    return tok, seg


# speedup vs baseline: 1.0200x; 1.0107x over previous
import jax
import jax.numpy as jnp
from jax import lax
from jax.experimental import pallas as pl
from jax.experimental.pallas import tpu as pltpu
from jax.experimental.pallas import tpu_sc as plsc

B = 16
S = 2048
LANES = 16
START_VALUE = 101
END_VALUE = 102
PAD_VALUE = 0
BUDGET = S - 3


def _body(seq_a, seq_b, len_a, len_b, tok_out, seg_out,
          src_v, lena_v, lenb_v, tok_v, seg_v, sem_seq, sem_len, sem_out):
    row = lax.axis_index("s")

    cpy_a = pltpu.async_copy(seq_a.at[row], src_v.at[pl.ds(0, S)], sem_seq)
    cpy_b = pltpu.async_copy(seq_b.at[row], src_v.at[pl.ds(S, S)], sem_seq)
    cpy_la = pltpu.async_copy(len_a, lena_v, sem_len)
    cpy_lb = pltpu.async_copy(len_b, lenb_v, sem_len)
    cpy_la.wait()
    cpy_lb.wait()

    rowv = jnp.full((LANES,), row, jnp.int32)
    l1 = jnp.minimum(plsc.load_gather(lena_v, [rowv]), S)
    l2 = jnp.minimum(plsc.load_gather(lenb_v, [rowv]), S)
    a1 = jnp.minimum(l1, jnp.maximum((BUDGET + 1) // 2, BUDGET - l2))
    a2 = jnp.minimum(l2, jnp.maximum(BUDGET // 2, BUDGET - l1))
    e1 = 1 + a1
    s2 = 2 + a1
    e2 = s2 + a2
    s2m = s2 - S

    iota = lax.iota(jnp.int32, 16)
    zeros = jnp.zeros((LANES,), jnp.int32)
    ones = jnp.full((LANES,), 1, jnp.int32)

    cpy_a.wait()
    cpy_b.wait()

    @plsc.parallel_loop(0, S // LANES, unroll=4)
    def _loop(i):
        pos = iota + i * LANES
        m2 = pos >= s2
        idx = jnp.where(m2, pos - s2m, pos - 1)
        g = plsc.load_gather(src_v, [jnp.maximum(idx, zeros)])
        gt2 = pos > e2
        tok = jnp.where(gt2, zeros, g)
        seg = jnp.where(m2 & jnp.logical_not(gt2), ones, zeros)
        tok_v[pl.ds(i * LANES, LANES)] = tok
        seg_v[pl.ds(i * LANES, LANES)] = seg

    cand = jnp.where(iota == 0, zeros, jnp.where(iota == 1, e1, e2))
    val = jnp.where(iota == 0, jnp.full((LANES,), START_VALUE, jnp.int32),
                    jnp.full((LANES,), END_VALUE, jnp.int32))
    mask = iota < 3
    plsc.store_scatter(tok_v, [jnp.clip(cand, 0, S - 1)], val, mask=mask)

    out_a = pltpu.async_copy(tok_v, tok_out.at[row], sem_out)
    out_b = pltpu.async_copy(seg_v, seg_out.at[row], sem_out)
    out_a.wait()
    out_b.wait()


@jax.jit
def kernel(seq_a, seq_b, len_a, len_b):
    mesh = plsc.VectorSubcoreMesh(core_axis_name="c", subcore_axis_name="s",
                                  num_cores=1, num_subcores=16)
    out_type = (jax.ShapeDtypeStruct((B, S), jnp.int32),
                jax.ShapeDtypeStruct((B, S), jnp.int32))
    scratch = [
        pltpu.VMEM((2 * S,), jnp.int32),
        pltpu.VMEM((B,), jnp.int32),
        pltpu.VMEM((B,), jnp.int32),
        pltpu.VMEM((S,), jnp.int32),
        pltpu.VMEM((S,), jnp.int32),
        pltpu.SemaphoreType.DMA,
        pltpu.SemaphoreType.DMA,
        pltpu.SemaphoreType.DMA,
    ]
    f = pl.kernel(_body, out_type=out_type, mesh=mesh, scratch_types=scratch,
                  compiler_params=pltpu.CompilerParams(needs_layout_passes=False))
    tok, seg = f(seq_a.astype(jnp.int32), seq_b.astype(jnp.int32),
                 len_a.astype(jnp.int32), len_b.astype(jnp.int32))
    return tok, seg
